# Initial kernel scaffold; baseline (speedup 1.0000x reference)
import jax
import jax.numpy as jnp
from jax.experimental import pallas as pl


def kernel(x, table):
    raise NotImplementedError("write your pallas kernel here")


# speedup vs baseline: 4.0416x; 4.0416x over previous
import functools

import jax
import jax.numpy as jnp
from jax import lax
from jax.experimental import pallas as pl
from jax.experimental.pallas import tpu as pltpu
from jax.experimental.pallas import tpu_sc as plsc

ROWS, COLS = 32, 8192
D = 64
NC, NS = 2, 16
NW = NC * NS
B_PER_W = ROWS * COLS // NW
CHUNK = 128
N_CHUNKS = B_PER_W // CHUNK
NBUF = 4
NG = N_CHUNKS // NBUF

_mesh = plsc.VectorSubcoreMesh(core_axis_name="c", subcore_axis_name="s")


@functools.partial(
    pl.kernel,
    mesh=_mesh,
    out_type=jax.ShapeDtypeStruct((ROWS, COLS, D), jnp.float32),
    scratch_types=[
        pltpu.VMEM((B_PER_W,), jnp.int32),
        pltpu.VMEM((NBUF, CHUNK, D), jnp.float32),
    ]
    + [pltpu.SemaphoreType.DMA] * (2 * NBUF),
    compiler_params=pltpu.CompilerParams(use_tc_tiling_on_sc=False),
)
def _gather_kernel(x_hbm, table_hbm, out_hbm, idx_v, bufs, *sems):
    gsems, osems = sems[:NBUF], sems[NBUF:]
    wid = lax.axis_index("s") * NC + lax.axis_index("c")
    pltpu.sync_copy(x_hbm.at[wid], idx_v)

    def idx_chunk(j):
        return idx_v.at[pl.ds(j * CHUNK, CHUNK)]

    for b in range(NBUF):
        pltpu.async_copy(table_hbm.at[idx_chunk(b)], bufs.at[b], gsems[b])

    def group_body(g, carry):
        base = g * NBUF
        for b in range(NBUF):
            j = base + b
            pltpu.make_async_copy(
                table_hbm.at[idx_chunk(j)], bufs.at[b], gsems[b]).wait()
            pltpu.async_copy(
                bufs.at[b], out_hbm.at[wid, pl.ds(j * CHUNK, CHUNK)],
                osems[b])

        @pl.when(g < NG - 1)
        def _refill():
            for b in range(NBUF):
                pltpu.make_async_copy(
                    bufs.at[b],
                    out_hbm.at[wid, pl.ds((base + b) * CHUNK, CHUNK)],
                    osems[b]).wait()
                pltpu.async_copy(
                    table_hbm.at[idx_chunk(base + NBUF + b)], bufs.at[b],
                    gsems[b])

        return carry

    lax.fori_loop(0, NG, group_body, 0)

    for b in range(NBUF):
        pltpu.make_async_copy(
            bufs.at[b], out_hbm.at[wid, pl.ds(b * CHUNK, CHUNK)],
            osems[b]).wait()


def kernel(x, table):
    return _gather_kernel(x, table)
